# baseline (speedup 1.0000x reference)
import functools

import jax
import jax.numpy as jnp
from jax import lax
from jax.experimental import pallas as pl
from jax.experimental.pallas import tpu as pltpu
from jax.experimental.pallas import tpu_sc as plsc

N = 16384
D = 1024
L = 16
C = 16
TBLK = 2048


def _mask_body(x_ref, w_ref, b_ref, u_ref, m_ref):
    x = x_ref[...]
    logits = lax.dot_general(
        x, w_ref[...],
        dimension_numbers=(((1,), (1,)), ((), ())),
        precision=lax.Precision.DEFAULT,
        preferred_element_type=jnp.float32,
    ) + b_ref[...]
    u = u_ref[...]
    g = -jnp.log(-jnp.log(u + 1e-10) + 1e-10)
    z = (logits + g) / 0.5
    m_ref[...] = (z[:, 1] > z[:, 0]).astype(jnp.float32)[:, None]


def _masks(x, W, b2, gumbel_u):
    return pl.pallas_call(
        _mask_body,
        grid=(N // TBLK,),
        in_specs=[
            pl.BlockSpec((TBLK, D), lambda i: (i, 0)),
            pl.BlockSpec((2, D), lambda i: (0, 0)),
            pl.BlockSpec((1, 2), lambda i: (0, 0)),
            pl.BlockSpec((TBLK, 2), lambda i: (i, 0)),
        ],
        out_specs=pl.BlockSpec((TBLK, 1), lambda i: (i, 0)),
        out_shape=jax.ShapeDtypeStruct((N, 1), jnp.float32),
    )(x, W, b2, gumbel_u)


def _sc_masked_copy(x, m):
    mesh = plsc.VectorSubcoreMesh(core_axis_name="c", subcore_axis_name="s")

    @functools.partial(
        pl.kernel,
        mesh=mesh,
        out_type=jax.ShapeDtypeStruct((N, D), jnp.float32),
        compiler_params=pltpu.CompilerParams(needs_layout_passes=False),
    )
    def sc_k(x_hbm, m_hbm, o_hbm):
        def body(x_in, m_in, o_out):
            mv = m_in[0]

            @pl.loop(0, C)
            def _(r):
                ridx = jnp.full((L,), r, jnp.int32)
                mr = lax.gather(
                    mv, ridx[:, None],
                    dimension_numbers=lax.GatherDimensionNumbers(
                        offset_dims=(), collapsed_slice_dims=(0,),
                        start_index_map=(0,)),
                    slice_sizes=(1,),
                    mode=lax.GatherScatterMode.PROMISE_IN_BOUNDS)

                @pl.loop(0, D, step=L)
                def _(c2):
                    o_out[r, pl.ds(c2, L)] = x_in[r, pl.ds(c2, L)] * mr

        pltpu.emit_pipeline(
            body,
            grid=(N // C,),
            in_specs=[
                pl.BlockSpec((C, D), lambda i: (i, 0)),
                pl.BlockSpec((1, C), lambda i: (i, 0)),
            ],
            out_specs=[pl.BlockSpec((C, D), lambda i: (i, 0))],
            core_axis_name=("c", "s"),
            dimension_semantics=(pltpu.PARALLEL,),
        )(x_hbm, m_hbm, o_hbm)

    return sc_k(x, m)


def kernel(x, W, b, gumbel_u):
    b2 = b.reshape(1, 2)
    m = _masks(x, W, b2, gumbel_u).reshape(N // C, C)
    return _sc_masked_copy(x, m)


# speedup vs baseline: 4.7629x; 4.7629x over previous
import jax
import jax.numpy as jnp
from jax.experimental import pallas as pl
from jax.experimental.pallas import tpu as pltpu

N = 16384
D = 1024
BLK = 2048
N_SPLIT = 4096


def _body(x_ref, w_ref, b_ref, u_ref, o_ref):
    x = x_ref[...]
    logits = jax.lax.dot_general(
        x, w_ref[...],
        dimension_numbers=(((1,), (1,)), ((), ())),
        precision=jax.lax.Precision.DEFAULT,
        preferred_element_type=jnp.float32,
    ) + b_ref[...]
    u = u_ref[...]
    g = -jnp.log(-jnp.log(u + 1e-10) + 1e-10)
    z = (logits + g) / 0.5
    mask = (z[:, 1] > z[:, 0]).astype(x.dtype)
    o_ref[...] = x * mask[:, None]


def kernel(x, W, b, gumbel_u):
    b2 = b.reshape(1, 2)
    return pl.pallas_call(
        _body,
        grid=(N // BLK,),
        in_specs=[
            pl.BlockSpec((BLK, D), lambda i: (i, 0)),
            pl.BlockSpec((2, D), lambda i: (0, 0)),
            pl.BlockSpec((1, 2), lambda i: (0, 0)),
            pl.BlockSpec((BLK, 2), lambda i: (i, 0)),
        ],
        out_specs=pl.BlockSpec((BLK, D), lambda i: (i, 0)),
        out_shape=jax.ShapeDtypeStruct((N, D), x.dtype),
        compiler_params=pltpu.CompilerParams(
            dimension_semantics=("parallel",),
        ),
    )(x, W, b2, gumbel_u)


# speedup vs baseline: 4.8848x; 1.0256x over previous
import jax
import jax.numpy as jnp
from jax.experimental import pallas as pl
from jax.experimental.pallas import tpu as pltpu

N = 16384
D = 1024
CH = 1024
NBUF = 4
NCH = N // CH


def _body(w_ref, b_ref, u_ref, x_hbm, o_hbm, xbuf, obuf, insem, outsem):
    def in_copy(k, b):
        return pltpu.make_async_copy(
            x_hbm.at[pl.ds(k * CH, CH), :], xbuf.at[b], insem.at[b])

    def out_copy(k, b):
        return pltpu.make_async_copy(
            obuf.at[b], o_hbm.at[pl.ds(k * CH, CH), :], outsem.at[b])

    for b in range(NBUF):
        in_copy(b, b).start()

    for k in range(NCH):
        b = k % NBUF
        in_copy(k, b).wait()
        x = xbuf[b]
        logits = jax.lax.dot_general(
            x, w_ref[...],
            dimension_numbers=(((1,), (1,)), ((), ())),
            precision=jax.lax.Precision.DEFAULT,
            preferred_element_type=jnp.float32,
        ) + b_ref[...]
        u = u_ref[pl.ds(k * CH, CH), :]
        g = -jnp.log(-jnp.log(u + 1e-10) + 1e-10)
        z = (logits + g) / 0.5
        mask = (z[:, 1] > z[:, 0]).astype(x.dtype)
        if k >= NBUF:
            out_copy(k - NBUF, b).wait()
        obuf[b] = x * mask[:, None]
        out_copy(k, b).start()
        if k + NBUF < NCH:
            in_copy(k + NBUF, b).start()

    for k in range(NCH - NBUF, NCH):
        out_copy(k, k % NBUF).wait()


def kernel(x, W, b, gumbel_u):
    b2 = b.reshape(1, 2)
    return pl.pallas_call(
        _body,
        in_specs=[
            pl.BlockSpec(memory_space=pltpu.VMEM),
            pl.BlockSpec(memory_space=pltpu.VMEM),
            pl.BlockSpec(memory_space=pltpu.VMEM),
            pl.BlockSpec(memory_space=pl.ANY),
        ],
        out_specs=pl.BlockSpec(memory_space=pl.ANY),
        out_shape=jax.ShapeDtypeStruct((N, D), x.dtype),
        scratch_shapes=[
            pltpu.VMEM((NBUF, CH, D), jnp.float32),
            pltpu.VMEM((NBUF, CH, D), jnp.float32),
            pltpu.SemaphoreType.DMA((NBUF,)),
            pltpu.SemaphoreType.DMA((NBUF,)),
        ],
    )(W, b2, gumbel_u, x)
